# baseline (speedup 1.0000x reference)
import functools

import jax
import jax.numpy as jnp
from jax.experimental import pallas as pl
from jax.experimental.pallas import tpu as pltpu

_SIM_T = 0.7
_VIRAL_T = 0.2
_K = 10
_CW = 16
_AW = 8


def _pick_nt(n):
    for c in (2000, 2048, 1024, 1000, 512, 256, 128, 64, 32, 16, 8):
        if n % c == 0:
            return c
    return n


def _insert(cval, cvir, ccnt, c16i, do, v, vir_s, cnt_s):
    c = cval[...]
    mn = jnp.min(c, axis=1, keepdims=True)
    do = do & (v > mn)
    colmn = jnp.min(jnp.where(c == mn, c16i, _CW), axis=1, keepdims=True)
    upd = (c16i == colmn) & do
    cval[...] = jnp.where(upd, v, c)
    cvir[...] = jnp.where(upd, vir_s, cvir[...])
    ccnt[...] = jnp.where(upd, cnt_s, ccnt[...])


def _knn_kernel(feats_ref, keys_ref, aux_ref, viral_ref, cnt_ref, out_ref,
                s_scr, cval, cvir, ccnt, rem, *, nt, n_tiles):
    n = pl.program_id(0)
    c16i = jax.lax.broadcasted_iota(jnp.int32, cval.shape, 1)
    neg = jnp.bfloat16(-1e9)

    @pl.when(n == 0)
    def _init():
        cval[...] = jnp.where(c16i < _K, jnp.float32(-1e9), jnp.float32(1e30))
        cvir[...] = jnp.zeros_like(cvir)
        ccnt[...] = jnp.zeros_like(ccnt)

    @pl.when(n > 0)
    def _process_prev():
        s = s_scr[...]
        mask = (s > _SIM_T).astype(jnp.bfloat16)
        aux = aux_ref[0]
        m = jax.lax.dot_general(mask, aux, (((1,), (0,)), ((), ())),
                                preferred_element_type=jnp.float32)
        count = m[:, 0:1]
        maxcnt = jnp.max(count)

        @pl.when((maxcnt > 0.5) & (maxcnt < 1.5))
        def _fast():
            v = jnp.max(s, axis=1, keepdims=True).astype(jnp.float32)
            _insert(cval, cvir, ccnt, c16i, count > 0.5, v,
                    m[:, 1:2], m[:, 2:3] + m[:, 3:4])

        @pl.when(maxcnt > 1.5)
        def _slow():
            rem[0] = maxcnt
            iota = jax.lax.broadcasted_iota(jnp.int32, s.shape, 1)
            vrow = viral_ref[0]
            crow = cnt_ref[0]
            for _ in range(_K):
                @pl.when(rem[0] > 0.5)
                def _one():
                    sk = s_scr[...]
                    skf = sk.astype(jnp.float32)
                    mvf = jnp.max(skf, axis=1, keepdims=True)
                    col = jnp.min(jnp.where(skf == mvf, iota, nt),
                                  axis=1, keepdims=True)
                    onec = iota == col
                    vir_s = jnp.sum(jnp.where(onec, vrow, 0.0),
                                    axis=1, keepdims=True)
                    cnt_s = jnp.sum(jnp.where(onec, crow, 0.0),
                                    axis=1, keepdims=True)
                    s_scr[...] = jnp.where(onec, neg, sk)
                    rmax = jnp.max(jnp.where(onec, jnp.float32(-1e9), skf),
                                   axis=1)
                    rem[0] = jnp.where(jnp.max(rmax) > _SIM_T, 1.0, 0.0)
                    _insert(cval, cvir, ccnt, c16i, mvf > _SIM_T, mvf,
                            vir_s, cnt_s)

    @pl.when(n < n_tiles)
    def _matmul():
        feats = feats_ref[...]
        keys = keys_ref[...].astype(jnp.bfloat16)
        s = jax.lax.dot_general(feats, keys, (((1,), (1,)), ((), ())),
                                preferred_element_type=jnp.float32)
        s_scr[...] = s.astype(jnp.bfloat16)

    @pl.when(n == n_tiles)
    def _finalize():
        vals = cval[...]
        vir = cvir[...]
        cnt = ccnt[...]
        keep = (vals > _SIM_T) & (c16i < _K)
        kv = keep & (vir > 0.5)
        nk = jnp.sum(keep.astype(jnp.float32), axis=1, keepdims=True)
        nv = jnp.sum(kv.astype(jnp.float32), axis=1, keepdims=True)
        mx = jnp.max(jnp.where(kv, vals, jnp.float32(-1e9)),
                     axis=1, keepdims=True)
        e = jnp.where(kv, jnp.exp(vals - mx), 0.0)
        z = jnp.sum(e, axis=1, keepdims=True)
        p = jnp.sum(e * cnt, axis=1, keepdims=True)
        pred = p / jnp.maximum(z, jnp.float32(1e-30))
        ratio = nv / jnp.maximum(nk, 1.0)
        cond = (nk > 0) & (ratio >= _VIRAL_T) & (nv > 0)
        out_ref[...] = jnp.where(cond, pred, 0.0)


@jax.jit
def kernel(feature_embedding, keys, if_viral, retweet_cnt):
    b, d = feature_embedding.shape
    n = keys.shape[0]
    nt = _pick_nt(n)
    n_tiles = n // nt
    viral_f = if_viral.astype(jnp.float32)
    cnt_f = retweet_cnt.astype(jnp.float32)
    cnt_hi = cnt_f.astype(jnp.bfloat16)
    cnt_lo = (cnt_f - cnt_hi.astype(jnp.float32)).astype(jnp.bfloat16)
    aux = jnp.concatenate(
        [jnp.ones((n, 1), jnp.bfloat16),
         viral_f[:, None].astype(jnp.bfloat16),
         cnt_hi[:, None], cnt_lo[:, None],
         jnp.zeros((n, _AW - 4), jnp.bfloat16)], axis=1,
    ).reshape(n_tiles, nt, _AW)
    viral3d = viral_f.reshape(n_tiles, 1, nt)
    cnt3d = cnt_f.reshape(n_tiles, 1, nt)

    last = n_tiles - 1
    out = pl.pallas_call(
        functools.partial(_knn_kernel, nt=nt, n_tiles=n_tiles),
        grid=(n_tiles + 1,),
        in_specs=[
            pl.BlockSpec((b, d), lambda i: (0, 0)),
            pl.BlockSpec((nt, d), lambda i: (jnp.minimum(i, last), 0)),
            pl.BlockSpec((1, nt, _AW), lambda i: (jnp.maximum(i - 1, 0), 0, 0)),
            pl.BlockSpec((1, 1, nt), lambda i: (jnp.maximum(i - 1, 0), 0, 0)),
            pl.BlockSpec((1, 1, nt), lambda i: (jnp.maximum(i - 1, 0), 0, 0)),
        ],
        out_specs=pl.BlockSpec((b, 1), lambda i: (0, 0)),
        out_shape=jax.ShapeDtypeStruct((b, 1), jnp.float32),
        scratch_shapes=[
            pltpu.VMEM((b, nt), jnp.bfloat16),
            pltpu.VMEM((b, _CW), jnp.float32),
            pltpu.VMEM((b, _CW), jnp.float32),
            pltpu.VMEM((b, _CW), jnp.float32),
            pltpu.SMEM((1,), jnp.float32),
        ],
        compiler_params=pltpu.CompilerParams(
            dimension_semantics=("arbitrary",),
            vmem_limit_bytes=63 * 1024 * 1024,
        ),
    )(feature_embedding.astype(jnp.bfloat16), keys, aux, viral3d, cnt3d)
    return out.reshape(b)


# speedup vs baseline: 1.1760x; 1.1760x over previous
import functools

import jax
import jax.numpy as jnp
from jax.experimental import pallas as pl
from jax.experimental.pallas import tpu as pltpu

_SIM_T = 0.7
_VIRAL_T = 0.2
_K = 10
_CW = 16
_AW = 8


def _pick_nt(n):
    for c in (2000, 2048, 1024, 1000, 512, 256, 128, 64, 32, 16, 8):
        if n % c == 0:
            return c
    return n


def _insert(cval, cvir, ccnt, c16i, do, v, vir_s, cnt_s):
    c = cval[...]
    mn = jnp.min(c, axis=1, keepdims=True)
    do = do & (v > mn)
    colmn = jnp.min(jnp.where(c == mn, c16i, _CW), axis=1, keepdims=True)
    upd = (c16i == colmn) & do
    cval[...] = jnp.where(upd, v, c)
    cvir[...] = jnp.where(upd, vir_s, cvir[...])
    ccnt[...] = jnp.where(upd, cnt_s, ccnt[...])


def _knn_kernel(feats_ref, keys_ref, aux_ref, viral_ref, cnt_ref, out_ref,
                s_scr, cval, cvir, ccnt, rem, *, nt, n_tiles):
    n = pl.program_id(0)
    c16i = jax.lax.broadcasted_iota(jnp.int32, cval.shape, 1)

    @pl.when(n == 0)
    def _init():
        cval[...] = jnp.where(c16i < _K, jnp.float32(-1e9), jnp.float32(1e30))
        cvir[...] = jnp.zeros_like(cvir)
        ccnt[...] = jnp.zeros_like(ccnt)

    @pl.when(n > 0)
    def _process_prev():
        s = s_scr[...]
        mask = (s > _SIM_T).astype(jnp.float32)
        aux = aux_ref[0]
        m = jax.lax.dot_general(mask, aux, (((1,), (0,)), ((), ())),
                                preferred_element_type=jnp.float32)
        count = m[:, 0:1]
        maxcnt = jnp.max(count)

        @pl.when((maxcnt > 0.5) & (maxcnt < 1.5))
        def _fast():
            v = jnp.max(s, axis=1, keepdims=True)
            _insert(cval, cvir, ccnt, c16i, count > 0.5, v,
                    m[:, 1:2], m[:, 2:3])

        @pl.when(maxcnt > 1.5)
        def _slow():
            rem[0] = maxcnt
            iota = jax.lax.broadcasted_iota(jnp.int32, s.shape, 1)
            vrow = viral_ref[0]
            crow = cnt_ref[0]
            for _ in range(_K):
                @pl.when(rem[0] > 0.5)
                def _one():
                    sk = s_scr[...]
                    mv = jnp.max(sk, axis=1, keepdims=True)
                    col = jnp.min(jnp.where(sk == mv, iota, nt),
                                  axis=1, keepdims=True)
                    onec = iota == col
                    vir_s = jnp.sum(jnp.where(onec, vrow, 0.0),
                                    axis=1, keepdims=True)
                    cnt_s = jnp.sum(jnp.where(onec, crow, 0.0),
                                    axis=1, keepdims=True)
                    smask = jnp.where(onec, jnp.float32(-1e9), sk)
                    s_scr[...] = smask
                    rem[0] = jnp.where(jnp.max(smask) > _SIM_T, 1.0, 0.0)
                    _insert(cval, cvir, ccnt, c16i, mv > _SIM_T, mv,
                            vir_s, cnt_s)

    @pl.when(n < n_tiles)
    def _matmul():
        feats = feats_ref[...]
        keys = keys_ref[...].astype(jnp.bfloat16)
        s = jax.lax.dot_general(feats, keys, (((1,), (1,)), ((), ())),
                                preferred_element_type=jnp.float32)
        s_scr[...] = s

    @pl.when(n == n_tiles)
    def _finalize():
        vals = cval[...]
        vir = cvir[...]
        cnt = ccnt[...]
        keep = (vals > _SIM_T) & (c16i < _K)
        kv = keep & (vir > 0.5)
        nk = jnp.sum(keep.astype(jnp.float32), axis=1, keepdims=True)
        nv = jnp.sum(kv.astype(jnp.float32), axis=1, keepdims=True)
        mx = jnp.max(jnp.where(kv, vals, jnp.float32(-1e9)),
                     axis=1, keepdims=True)
        e = jnp.where(kv, jnp.exp(vals - mx), 0.0)
        z = jnp.sum(e, axis=1, keepdims=True)
        p = jnp.sum(e * cnt, axis=1, keepdims=True)
        pred = p / jnp.maximum(z, jnp.float32(1e-30))
        ratio = nv / jnp.maximum(nk, 1.0)
        cond = (nk > 0) & (ratio >= _VIRAL_T) & (nv > 0)
        out_ref[...] = jnp.where(cond, pred, 0.0)


@jax.jit
def kernel(feature_embedding, keys, if_viral, retweet_cnt):
    b, d = feature_embedding.shape
    n = keys.shape[0]
    nt = _pick_nt(n)
    n_tiles = n // nt
    viral_f = if_viral.astype(jnp.float32)
    cnt_f = retweet_cnt.astype(jnp.float32)
    aux = jnp.concatenate(
        [jnp.ones((n, 1), jnp.float32), viral_f[:, None], cnt_f[:, None],
         jnp.zeros((n, _AW - 3), jnp.float32)], axis=1,
    ).reshape(n_tiles, nt, _AW)
    viral3d = viral_f.reshape(n_tiles, 1, nt)
    cnt3d = cnt_f.reshape(n_tiles, 1, nt)

    last = n_tiles - 1
    out = pl.pallas_call(
        functools.partial(_knn_kernel, nt=nt, n_tiles=n_tiles),
        grid=(n_tiles + 1,),
        in_specs=[
            pl.BlockSpec((b, d), lambda i: (0, 0)),
            pl.BlockSpec((nt, d), lambda i: (jnp.minimum(i, last), 0)),
            pl.BlockSpec((1, nt, _AW), lambda i: (jnp.maximum(i - 1, 0), 0, 0)),
            pl.BlockSpec((1, 1, nt), lambda i: (jnp.maximum(i - 1, 0), 0, 0)),
            pl.BlockSpec((1, 1, nt), lambda i: (jnp.maximum(i - 1, 0), 0, 0)),
        ],
        out_specs=pl.BlockSpec((b, 1), lambda i: (0, 0)),
        out_shape=jax.ShapeDtypeStruct((b, 1), jnp.float32),
        scratch_shapes=[
            pltpu.VMEM((b, nt), jnp.float32),
            pltpu.VMEM((b, _CW), jnp.float32),
            pltpu.VMEM((b, _CW), jnp.float32),
            pltpu.VMEM((b, _CW), jnp.float32),
            pltpu.SMEM((1,), jnp.float32),
        ],
        compiler_params=pltpu.CompilerParams(
            dimension_semantics=("arbitrary",),
            vmem_limit_bytes=63 * 1024 * 1024,
        ),
    )(feature_embedding.astype(jnp.bfloat16), keys, aux, viral3d, cnt3d)
    return out.reshape(b)


# speedup vs baseline: 1.1785x; 1.0021x over previous
import functools

import jax
import jax.numpy as jnp
from jax.experimental import pallas as pl
from jax.experimental.pallas import tpu as pltpu

_SIM_T = 0.7
_VIRAL_T = 0.2
_K = 10
_CW = 16
_AW = 8


def _pick_nt(n):
    for c in (2000, 2048, 1024, 1000, 512, 256, 128, 64, 32, 16, 8):
        if n % c == 0:
            return c
    return n


def _insert(cval, cvir, ccnt, c16i, do, v, vir_s, cnt_s):
    c = cval[...]
    mn = jnp.min(c, axis=1, keepdims=True)
    do = do & (v > mn)
    colmn = jnp.min(jnp.where(c == mn, c16i, _CW), axis=1, keepdims=True)
    upd = (c16i == colmn) & do
    cval[...] = jnp.where(upd, v, c)
    cvir[...] = jnp.where(upd, vir_s, cvir[...])
    ccnt[...] = jnp.where(upd, cnt_s, ccnt[...])


def _knn_kernel(feats_ref, keys_ref, aux_ref, viral_ref, cnt_ref, out_ref,
                s_scr, cval, cvir, ccnt, rem, *, nt, n_tiles):
    n = pl.program_id(0)
    c16i = jax.lax.broadcasted_iota(jnp.int32, cval.shape, 1)

    @pl.when(n == 0)
    def _init():
        cval[...] = jnp.where(c16i < _K, jnp.float32(-1e9), jnp.float32(1e30))
        cvir[...] = jnp.zeros_like(cvir)
        ccnt[...] = jnp.zeros_like(ccnt)

    @pl.when(n > 0)
    def _process_prev():
        s = s_scr[...]
        mask = (s > _SIM_T).astype(jnp.float32)
        aux = aux_ref[0]
        m = jax.lax.dot_general(mask, aux, (((1,), (0,)), ((), ())),
                                preferred_element_type=jnp.float32)
        count = m[:, 0:1]
        maxcnt = jnp.max(count)

        @pl.when((maxcnt > 0.5) & (maxcnt < 1.5))
        def _fast():
            v = jnp.max(s, axis=1, keepdims=True)
            _insert(cval, cvir, ccnt, c16i, count > 0.5, v,
                    m[:, 1:2], m[:, 2:3])

        @pl.when(maxcnt > 1.5)
        def _slow():
            rem[0] = maxcnt
            iota = jax.lax.broadcasted_iota(jnp.int32, s.shape, 1)
            vrow = viral_ref[0]
            crow = cnt_ref[0]
            for _ in range(_K):
                @pl.when(rem[0] > 0.5)
                def _one():
                    sk = s_scr[...]
                    mv = jnp.max(sk, axis=1, keepdims=True)
                    col = jnp.min(jnp.where(sk == mv, iota, nt),
                                  axis=1, keepdims=True)
                    onec = iota == col
                    vir_s = jnp.sum(jnp.where(onec, vrow, 0.0),
                                    axis=1, keepdims=True)
                    cnt_s = jnp.sum(jnp.where(onec, crow, 0.0),
                                    axis=1, keepdims=True)
                    smask = jnp.where(onec, jnp.float32(-1e9), sk)
                    s_scr[...] = smask
                    rem[0] = jnp.where(jnp.max(smask) > _SIM_T, 1.0, 0.0)
                    _insert(cval, cvir, ccnt, c16i, mv > _SIM_T, mv,
                            vir_s, cnt_s)

    @pl.when(n < n_tiles)
    def _matmul():
        feats = feats_ref[...]
        keys = keys_ref[...]
        s = jax.lax.dot_general(feats, keys, (((1,), (1,)), ((), ())),
                                preferred_element_type=jnp.float32)
        s_scr[...] = s

    @pl.when(n == n_tiles)
    def _finalize():
        vals = cval[...]
        vir = cvir[...]
        cnt = ccnt[...]
        keep = (vals > _SIM_T) & (c16i < _K)
        kv = keep & (vir > 0.5)
        nk = jnp.sum(keep.astype(jnp.float32), axis=1, keepdims=True)
        nv = jnp.sum(kv.astype(jnp.float32), axis=1, keepdims=True)
        mx = jnp.max(jnp.where(kv, vals, jnp.float32(-1e9)),
                     axis=1, keepdims=True)
        e = jnp.where(kv, jnp.exp(vals - mx), 0.0)
        z = jnp.sum(e, axis=1, keepdims=True)
        p = jnp.sum(e * cnt, axis=1, keepdims=True)
        pred = p / jnp.maximum(z, jnp.float32(1e-30))
        ratio = nv / jnp.maximum(nk, 1.0)
        cond = (nk > 0) & (ratio >= _VIRAL_T) & (nv > 0)
        out_ref[...] = jnp.where(cond, pred, 0.0)


@jax.jit
def kernel(feature_embedding, keys, if_viral, retweet_cnt):
    b, d = feature_embedding.shape
    n = keys.shape[0]
    nt = _pick_nt(n)
    n_tiles = n // nt
    viral_f = if_viral.astype(jnp.float32)
    cnt_f = retweet_cnt.astype(jnp.float32)
    aux = jnp.concatenate(
        [jnp.ones((n, 1), jnp.float32), viral_f[:, None], cnt_f[:, None],
         jnp.zeros((n, _AW - 3), jnp.float32)], axis=1,
    ).reshape(n_tiles, nt, _AW)
    viral3d = viral_f.reshape(n_tiles, 1, nt)
    cnt3d = cnt_f.reshape(n_tiles, 1, nt)

    last = n_tiles - 1
    out = pl.pallas_call(
        functools.partial(_knn_kernel, nt=nt, n_tiles=n_tiles),
        grid=(n_tiles + 1,),
        in_specs=[
            pl.BlockSpec((b, d), lambda i: (0, 0)),
            pl.BlockSpec((nt, d), lambda i: (jnp.minimum(i, last), 0)),
            pl.BlockSpec((1, nt, _AW), lambda i: (jnp.maximum(i - 1, 0), 0, 0)),
            pl.BlockSpec((1, 1, nt), lambda i: (jnp.maximum(i - 1, 0), 0, 0)),
            pl.BlockSpec((1, 1, nt), lambda i: (jnp.maximum(i - 1, 0), 0, 0)),
        ],
        out_specs=pl.BlockSpec((b, 1), lambda i: (0, 0)),
        out_shape=jax.ShapeDtypeStruct((b, 1), jnp.float32),
        scratch_shapes=[
            pltpu.VMEM((b, nt), jnp.float32),
            pltpu.VMEM((b, _CW), jnp.float32),
            pltpu.VMEM((b, _CW), jnp.float32),
            pltpu.VMEM((b, _CW), jnp.float32),
            pltpu.SMEM((1,), jnp.float32),
        ],
        compiler_params=pltpu.CompilerParams(
            dimension_semantics=("arbitrary",),
            vmem_limit_bytes=63 * 1024 * 1024,
        ),
    )(feature_embedding.astype(jnp.float32), keys, aux, viral3d, cnt3d)
    return out.reshape(b)
